# baseline (speedup 1.0000x reference)
import functools

import jax
import jax.numpy as jnp
from jax import lax
from jax.experimental import pallas as pl
from jax.experimental.pallas import tpu as pltpu
from jax.experimental.pallas import tpu_sc as plsc

N = 10000
NP = 10240
E = 320000
D = 128

NC = 2
NS = 16
NW = NC * NS
L = 16
EPT = E // NW
CH = 80
IT = EPT // CH
DH = D // NC
EPS = E // NS
IT2 = EPS // CH
STRIPE = NP // NS
NR = NP // 128

_mesh = plsc.VectorSubcoreMesh(core_axis_name="c", subcore_axis_name="s")


def _deg_body(eidx_h, z80_h, idn_h, out_h,
              srcv, dstv, dego_v, degi_v, idn_v, dego_sh, degi_sh):
    c = lax.axis_index("c")
    s = lax.axis_index("s")
    wid = c * NS + s
    half = pl.ds((wid % 2) * IT, IT)
    pltpu.sync_copy(eidx_h.at[0, wid // 2, half], srcv)
    pltpu.sync_copy(eidx_h.at[2, wid // 2, half], dstv)
    pltpu.sync_copy(idn_h, idn_v)
    pltpu.sync_copy(z80_h, dego_v)
    pltpu.sync_copy(z80_h, degi_v)
    zs = pl.ds(s * (NR // NS), NR // NS)
    pltpu.sync_copy(z80_h.at[pl.ds(0, NR // NS)], dego_sh.at[zs])
    pltpu.sync_copy(z80_h.at[pl.ds(0, NR // NS)], degi_sh.at[zs])
    plsc.subcore_barrier()
    ones16 = jnp.full((L,), 1.0, jnp.float32)
    c127 = jnp.full((L,), 127, jnp.int32)

    def chunk(j, carry):
        for k in range(CH // L):
            sl = pl.ds(k * L, L)
            sv = srcv[j, sl]
            dv = dstv[j, sl]
            plsc.addupdate_scatter(
                dego_v,
                [lax.shift_right_logical(sv, 8),
                 lax.shift_right_logical(sv, 1) & c127], ones16)
            plsc.addupdate_scatter(
                degi_v, [lax.shift_right_logical(dv, 7), dv & c127], ones16)
        return carry

    lax.fori_loop(0, IT, chunk, 0)
    pltpu.sync_copy(dego_v, dego_sh.at[idn_v], add=True)
    pltpu.sync_copy(degi_v, degi_sh.at[idn_v], add=True)
    plsc.subcore_barrier()
    pltpu.sync_copy(dego_sh.at[zs], out_h.at[c, 0, zs])
    pltpu.sync_copy(degi_sh.at[zs], out_h.at[c, 1, zs])


@functools.partial(
    pl.kernel,
    out_type=jax.ShapeDtypeStruct((NC, 2, NR, 128), jnp.float32),
    mesh=_mesh,
    compiler_params=pltpu.CompilerParams(
        needs_layout_passes=False, use_tc_tiling_on_sc=False),
    scratch_types=[
        pltpu.VMEM((IT, CH), jnp.int32),
        pltpu.VMEM((IT, CH), jnp.int32),
        pltpu.VMEM((NR, 128), jnp.float32),
        pltpu.VMEM((NR, 128), jnp.float32),
        pltpu.VMEM((NR,), jnp.int32),
        pltpu.VMEM_SHARED((NR, 128), jnp.float32),
        pltpu.VMEM_SHARED((NR, 128), jnp.float32),
    ],
)
def _deg_kernel(eidx_h, z80_h, idn_h, out_h, *scratch):
    _deg_body(eidx_h, z80_h, idn_h, out_h, *scratch)


G = 2
NG = IT2 // G


def _agg_body(m_h, nd_h, eidx_h, zrow_h, znp_h, agg_out, c_out,
              idxv, rbuf2, tbl, agg_sh, sg0, sg1, ss0, ss1):
    c = lax.axis_index("c")
    s = lax.axis_index("s")
    wid = c * NS + s
    @pl.when(c == 0)
    def _():
        pltpu.sync_copy(eidx_h.at[0, s], idxv.at[pl.ds(0, IT2)])

    @pl.when(c == 1)
    def _():
        pltpu.sync_copy(eidx_h.at[1, s], idxv.at[pl.ds(0, IT2)])

    pltpu.sync_copy(eidx_h.at[2, s], idxv.at[pl.ds(IT2, IT2)])
    pltpu.sync_copy(nd_h, tbl.at[pl.ds(0, NP)])
    pltpu.sync_copy(znp_h, tbl.at[pl.ds(NP, NP)])
    pltpu.sync_copy(zrow_h, agg_sh.at[pl.ds(s * STRIPE, STRIPE)])
    plsc.subcore_barrier()

    mc = m_h
    npv = jnp.full((L,), NP, jnp.int32)
    GB = G * CH

    def rb(p, k):
        return rbuf2.at[pl.ds((p * G + k) * CH, CH)]

    def fire_gathers(g, p, sem):
        for k in range(G):
            pltpu.async_copy(mc.at[idxv.at[g * G + k]], rb(p, k), sem)

    def drain_group(p, sem):
        pltpu.make_async_copy(
            mc.at[pl.ds(0, GB)], rbuf2.at[pl.ds(p * GB, GB)], sem).wait()

    def fire_scatters(g, p, sem):
        for k in range(G):
            pltpu.async_copy(
                rb(p, k), agg_sh.at[idxv.at[IT2 + g * G + k]], sem, add=True)

    def vpu_c(g):
        for k in range(G):
            j = g * G + k
            for kk in range(CH // L):
                sl = pl.ds(kk * L, L)
                vals = plsc.load_gather(tbl, [idxv[IT2 + j, sl]])
                plsc.addupdate_scatter(
                    tbl,
                    [lax.shift_right_logical(idxv[j, sl], 1) + npv], vals)

    fire_gathers(0, 0, sg0)

    def outer(i, carry):
        g0 = 2 * i
        fire_gathers(g0 + 1, 1, sg1)
        drain_group(0, sg0)
        fire_scatters(g0, 0, ss0)
        vpu_c(g0)
        drain_group(0, ss0)
        fire_gathers(g0 + 2, 0, sg0)
        drain_group(1, sg1)
        fire_scatters(g0 + 1, 1, ss1)
        vpu_c(g0 + 1)
        drain_group(1, ss1)
        return carry

    lax.fori_loop(0, NG // 2, outer, 0)
    drain_group(0, sg0)
    fire_scatters(NG - 1, 0, ss0)
    vpu_c(NG - 1)
    drain_group(0, ss0)

    pltpu.sync_copy(tbl.at[pl.ds(NP, NP)], c_out.at[wid])
    plsc.subcore_barrier()
    sl = pl.ds(s * STRIPE, STRIPE)
    pltpu.sync_copy(agg_sh.at[sl], agg_out.at[c, sl])


@functools.partial(
    pl.kernel,
    out_type=(
        jax.ShapeDtypeStruct((NC, NP, DH), jnp.float32),
        jax.ShapeDtypeStruct((NW, NP), jnp.float32),
    ),
    mesh=_mesh,
    compiler_params=pltpu.CompilerParams(
        needs_layout_passes=False, use_tc_tiling_on_sc=False),
    scratch_types=[
        pltpu.VMEM((2 * IT2, CH), jnp.int32),
        pltpu.VMEM((2 * G * CH, DH), jnp.float32),
        pltpu.VMEM((2 * NP,), jnp.float32),
        pltpu.VMEM_SHARED((NP, DH), jnp.float32),
        pltpu.SemaphoreType.DMA,
        pltpu.SemaphoreType.DMA,
        pltpu.SemaphoreType.DMA,
        pltpu.SemaphoreType.DMA,
    ],
)
def _agg_kernel(m_h, nd_h, eidx_h, zrow_h, znp_h,
                agg_out, c_out, *scratch):
    _agg_body(m_h, nd_h, eidx_h, zrow_h, znp_h,
              agg_out, c_out, *scratch)


def _mm_body(h_ref, w_ref, ns_ref, o_ref, sh_ref):
    o_ref[...] = (
        jnp.dot(h_ref[...], w_ref[...], preferred_element_type=jnp.float32)
        * ns_ref[...]
    )
    sh_ref[...] = jnp.sum(h_ref[...], axis=0, keepdims=True)


def _fin_body(agg_ref, c_ref, nd_ref, ns_ref, sh_ref, b1_ref,
              w2_ref, b2_ref, wl_ref, bl_ref, o_ref):
    agg = jnp.concatenate(
        [agg_ref[0, :N, :], agg_ref[1, :N, :]], axis=1)
    x = jnp.maximum(agg * nd_ref[...] + b1_ref[...], 0.0)
    c_row = 0.5 * jnp.sum(c_ref[...], axis=0, keepdims=True)[:, :N]
    w_row = ns_ref[...] * c_row
    s = jnp.dot(w_row, x, preferred_element_type=jnp.float32)
    sh = sh_ref[...]
    o_ref[...] = (
        jnp.dot(s, w2_ref[...], preferred_element_type=jnp.float32)
        + jnp.dot(sh, wl_ref[...], preferred_element_type=jnp.float32)
    ) * (1.0 / N) + b2_ref[...] + bl_ref[...]


def kernel(h, edge_index, W1, b1, W2, b2, Wl, bl):
    src = edge_index[0].astype(jnp.int32)
    dst = edge_index[1].astype(jnp.int32)
    s2 = src + src
    eidx = jnp.stack([s2, s2 + 1, dst]).reshape(3, NS, IT2, CH)
    zrow = jnp.zeros((STRIPE, DH), jnp.float32)
    znp = jnp.zeros((NP,), jnp.float32)
    z80 = jnp.zeros((NR, 128), jnp.float32)
    idn = jnp.arange(NR, dtype=jnp.int32)

    deg_parts = _deg_kernel(eidx, z80, idn)
    deg = deg_parts.sum(axis=0).reshape(2, NP)[:, :N]
    norm = jnp.where(deg > 0, lax.rsqrt(jnp.maximum(deg, 1.0)), 0.0)
    ns_col = norm[0].reshape(N, 1)
    nd_col = norm[1].reshape(N, 1)
    nd_flat = jnp.pad(norm[1], (0, NP - N))

    m, sh = pl.pallas_call(
        _mm_body,
        out_shape=(
            jax.ShapeDtypeStruct((N, D), jnp.float32),
            jax.ShapeDtypeStruct((1, D), jnp.float32),
        ),
    )(h, W1, ns_col)
    m2 = m.reshape(2 * N, DH)

    agg_pp, c_pp = _agg_kernel(m2, nd_flat, eidx, zrow, znp)
    ns_row = norm[0].reshape(1, N)

    out = pl.pallas_call(
        _fin_body,
        out_shape=jax.ShapeDtypeStruct((1, D), jnp.float32),
    )(agg_pp, c_pp, nd_col, ns_row, sh,
      b1.reshape(1, D), W2, b2.reshape(1, D), Wl, bl.reshape(1, D))
    return out


# speedup vs baseline: 1.0843x; 1.0843x over previous
import functools

import jax
import jax.numpy as jnp
from jax import lax
from jax.experimental import pallas as pl
from jax.experimental.pallas import tpu as pltpu
from jax.experimental.pallas import tpu_sc as plsc

N = 10000
NP = 10240
E = 320000
D = 128

NC = 2
NS = 16
NW = NC * NS
L = 16
EPT = E // NW
CH = 80
IT = EPT // CH
DH = D // NC
EPS = E // NS
IT2 = EPS // CH
STRIPE = NP // NS
NR = NP // 128

_mesh = plsc.VectorSubcoreMesh(core_axis_name="c", subcore_axis_name="s")


def _deg_body(s2a_h, dstf_h, z80_h, idn_h, out_h,
              srcv, dstv, dego_v, degi_v, idn_v, dego_sh, degi_sh):
    c = lax.axis_index("c")
    s = lax.axis_index("s")
    wid = c * NS + s
    pltpu.sync_copy(s2a_h.at[pl.ds(wid * EPT, EPT)], srcv)
    pltpu.sync_copy(dstf_h.at[pl.ds(wid * EPT, EPT)], dstv)
    pltpu.sync_copy(idn_h, idn_v)
    pltpu.sync_copy(z80_h, dego_v)
    pltpu.sync_copy(z80_h, degi_v)
    zs = pl.ds(s * (NR // NS), NR // NS)
    pltpu.sync_copy(z80_h.at[pl.ds(0, NR // NS)], dego_sh.at[zs])
    pltpu.sync_copy(z80_h.at[pl.ds(0, NR // NS)], degi_sh.at[zs])
    plsc.subcore_barrier()
    ones16 = jnp.full((L,), 1.0, jnp.float32)
    c127 = jnp.full((L,), 127, jnp.int32)

    def chunk(j, carry):
        for k in range(CH // L):
            sl = pl.ds(j * CH + k * L, L)
            sv = srcv[sl]
            dv = dstv[sl]
            plsc.addupdate_scatter(
                dego_v,
                [lax.shift_right_logical(sv, 8),
                 lax.shift_right_logical(sv, 1) & c127], ones16)
            plsc.addupdate_scatter(
                degi_v, [lax.shift_right_logical(dv, 7), dv & c127], ones16)
        return carry

    lax.fori_loop(0, IT, chunk, 0)
    pltpu.sync_copy(dego_v, dego_sh.at[idn_v], add=True)
    pltpu.sync_copy(degi_v, degi_sh.at[idn_v], add=True)
    plsc.subcore_barrier()
    pltpu.sync_copy(dego_sh.at[zs], out_h.at[c, 0, zs])
    pltpu.sync_copy(degi_sh.at[zs], out_h.at[c, 1, zs])


@functools.partial(
    pl.kernel,
    out_type=jax.ShapeDtypeStruct((NC, 2, NR, 128), jnp.float32),
    mesh=_mesh,
    compiler_params=pltpu.CompilerParams(
        needs_layout_passes=False, use_tc_tiling_on_sc=False),
    scratch_types=[
        pltpu.VMEM((EPT,), jnp.int32),
        pltpu.VMEM((EPT,), jnp.int32),
        pltpu.VMEM((NR, 128), jnp.float32),
        pltpu.VMEM((NR, 128), jnp.float32),
        pltpu.VMEM((NR,), jnp.int32),
        pltpu.VMEM_SHARED((NR, 128), jnp.float32),
        pltpu.VMEM_SHARED((NR, 128), jnp.float32),
    ],
)
def _deg_kernel(s2a_h, dstf_h, z80_h, idn_h, out_h, *scratch):
    _deg_body(s2a_h, dstf_h, z80_h, idn_h, out_h, *scratch)


G = 2
NG = IT2 // G


def _agg_body(m_h, nd_h, s2a_h, s2b_h, dstf_h, zrow_h, znp_h, agg_out, c_out,
              idxv, rbuf2, tbl, agg_sh, sg0, sg1, ss0, ss1):
    c = lax.axis_index("c")
    s = lax.axis_index("s")
    wid = c * NS + s
    es = pl.ds(s * EPS, EPS)

    @pl.when(c == 0)
    def _():
        pltpu.sync_copy(s2a_h.at[es], idxv.at[pl.ds(0, EPS)])

    @pl.when(c == 1)
    def _():
        pltpu.sync_copy(s2b_h.at[es], idxv.at[pl.ds(0, EPS)])

    pltpu.sync_copy(dstf_h.at[es], idxv.at[pl.ds(EPS, EPS)])
    pltpu.sync_copy(nd_h, tbl.at[pl.ds(0, NP)])
    pltpu.sync_copy(znp_h, tbl.at[pl.ds(NP, NP)])
    pltpu.sync_copy(zrow_h, agg_sh.at[pl.ds(s * STRIPE, STRIPE)])
    plsc.subcore_barrier()

    mc = m_h
    npv = jnp.full((L,), NP, jnp.int32)
    GB = G * CH

    def rb(p, k):
        return rbuf2.at[pl.ds((p * G + k) * CH, CH)]

    def gidx(j):
        return idxv.at[pl.ds(j * CH, CH)]

    def didx(j):
        return idxv.at[pl.ds(EPS + j * CH, CH)]

    def fire_gathers(g, p, sem):
        for k in range(G):
            pltpu.async_copy(mc.at[gidx(g * G + k)], rb(p, k), sem)

    def drain_group(p, sem):
        pltpu.make_async_copy(
            mc.at[pl.ds(0, GB)], rbuf2.at[pl.ds(p * GB, GB)], sem).wait()

    def fire_scatters(g, p, sem):
        for k in range(G):
            pltpu.async_copy(
                rb(p, k), agg_sh.at[didx(g * G + k)], sem, add=True)

    def vpu_c(g):
        for k in range(G):
            j = g * G + k
            for kk in range(CH // L):
                vals = plsc.load_gather(
                    tbl, [idxv[pl.ds(EPS + j * CH + kk * L, L)]])
                plsc.addupdate_scatter(
                    tbl,
                    [lax.shift_right_logical(
                        idxv[pl.ds(j * CH + kk * L, L)], 1) + npv], vals)

    fire_gathers(0, 0, sg0)

    def outer(i, carry):
        g0 = 2 * i
        fire_gathers(g0 + 1, 1, sg1)
        drain_group(0, sg0)
        fire_scatters(g0, 0, ss0)
        vpu_c(g0)
        drain_group(0, ss0)
        fire_gathers(g0 + 2, 0, sg0)
        drain_group(1, sg1)
        fire_scatters(g0 + 1, 1, ss1)
        vpu_c(g0 + 1)
        drain_group(1, ss1)
        return carry

    lax.fori_loop(0, NG // 2, outer, 0)
    drain_group(0, sg0)
    fire_scatters(NG - 1, 0, ss0)
    vpu_c(NG - 1)
    drain_group(0, ss0)

    pltpu.sync_copy(tbl.at[pl.ds(NP, NP)], c_out.at[wid])
    plsc.subcore_barrier()
    sl = pl.ds(s * STRIPE, STRIPE)
    pltpu.sync_copy(agg_sh.at[sl], agg_out.at[c, sl])


@functools.partial(
    pl.kernel,
    out_type=(
        jax.ShapeDtypeStruct((NC, NP, DH), jnp.float32),
        jax.ShapeDtypeStruct((NW, NP), jnp.float32),
    ),
    mesh=_mesh,
    compiler_params=pltpu.CompilerParams(
        needs_layout_passes=False, use_tc_tiling_on_sc=False),
    scratch_types=[
        pltpu.VMEM((2 * EPS,), jnp.int32),
        pltpu.VMEM((2 * G * CH, DH), jnp.float32),
        pltpu.VMEM((2 * NP,), jnp.float32),
        pltpu.VMEM_SHARED((NP, DH), jnp.float32),
        pltpu.SemaphoreType.DMA,
        pltpu.SemaphoreType.DMA,
        pltpu.SemaphoreType.DMA,
        pltpu.SemaphoreType.DMA,
    ],
)
def _agg_kernel(m_h, nd_h, s2a_h, s2b_h, dstf_h, zrow_h, znp_h,
                agg_out, c_out, *scratch):
    _agg_body(m_h, nd_h, s2a_h, s2b_h, dstf_h, zrow_h, znp_h,
              agg_out, c_out, *scratch)


def _mm_body(h_ref, w_ref, ns_ref, o_ref, sh_ref):
    o_ref[...] = (
        jnp.dot(h_ref[...], w_ref[...], preferred_element_type=jnp.float32)
        * ns_ref[...]
    )
    sh_ref[...] = jnp.sum(h_ref[...], axis=0, keepdims=True)


def _fin_body(agg_ref, c_ref, nd_ref, ns_ref, sh_ref, b1_ref,
              w2_ref, b2_ref, wl_ref, bl_ref, o_ref):
    agg = jnp.concatenate(
        [agg_ref[0, :N, :], agg_ref[1, :N, :]], axis=1)
    x = jnp.maximum(agg * nd_ref[...] + b1_ref[...], 0.0)
    c_row = 0.5 * jnp.sum(c_ref[...], axis=0, keepdims=True)[:, :N]
    w_row = ns_ref[...] * c_row
    s = jnp.dot(w_row, x, preferred_element_type=jnp.float32)
    sh = sh_ref[...]
    o_ref[...] = (
        jnp.dot(s, w2_ref[...], preferred_element_type=jnp.float32)
        + jnp.dot(sh, wl_ref[...], preferred_element_type=jnp.float32)
    ) * (1.0 / N) + b2_ref[...] + bl_ref[...]


def kernel(h, edge_index, W1, b1, W2, b2, Wl, bl):
    src = edge_index[0].astype(jnp.int32)
    dst = edge_index[1].astype(jnp.int32)
    s2a = src + src
    s2b = s2a + 1
    zrow = jnp.zeros((STRIPE, DH), jnp.float32)
    znp = jnp.zeros((NP,), jnp.float32)
    z80 = jnp.zeros((NR, 128), jnp.float32)
    idn = jnp.arange(NR, dtype=jnp.int32)

    deg_parts = _deg_kernel(s2a, dst, z80, idn)
    deg = deg_parts.sum(axis=0).reshape(2, NP)[:, :N]
    norm = jnp.where(deg > 0, lax.rsqrt(jnp.maximum(deg, 1.0)), 0.0)
    ns_col = norm[0].reshape(N, 1)
    nd_col = norm[1].reshape(N, 1)
    nd_flat = jnp.pad(norm[1], (0, NP - N))

    m, sh = pl.pallas_call(
        _mm_body,
        out_shape=(
            jax.ShapeDtypeStruct((N, D), jnp.float32),
            jax.ShapeDtypeStruct((1, D), jnp.float32),
        ),
    )(h, W1, ns_col)
    m2 = m.reshape(2 * N, DH)

    agg_pp, c_pp = _agg_kernel(m2, nd_flat, s2a, s2b, dst, zrow, znp)
    ns_row = norm[0].reshape(1, N)

    out = pl.pallas_call(
        _fin_body,
        out_shape=jax.ShapeDtypeStruct((1, D), jnp.float32),
    )(agg_pp, c_pp, nd_col, ns_row, sh,
      b1.reshape(1, D), W2, b2.reshape(1, D), Wl, bl.reshape(1, D))
    return out


# speedup vs baseline: 1.1340x; 1.0458x over previous
import functools

import jax
import jax.numpy as jnp
from jax import lax
from jax.experimental import pallas as pl
from jax.experimental.pallas import tpu as pltpu
from jax.experimental.pallas import tpu_sc as plsc

N = 10000
NP = 10240
E = 320000
D = 128

NC = 2
NS = 16
NW = NC * NS
L = 16
EPT = E // NW
CH = 80
IT = EPT // CH
DH = D // NC
EPS = E // NS
IT2 = EPS // CH
STRIPE = NP // NS
NR = NP // 128

_mesh = plsc.VectorSubcoreMesh(core_axis_name="c", subcore_axis_name="s")


def _deg_body(ei_h, z80_h, idn_h, out_h,
              srcv, dstv, dego_v, degi_v, idn_v, dego_sh, degi_sh):
    c = lax.axis_index("c")
    s = lax.axis_index("s")
    wid = c * NS + s
    pltpu.sync_copy(ei_h.at[0, pl.ds(wid * EPT, EPT)], srcv)
    pltpu.sync_copy(ei_h.at[1, pl.ds(wid * EPT, EPT)], dstv)
    pltpu.sync_copy(idn_h, idn_v)
    pltpu.sync_copy(z80_h, dego_v)
    pltpu.sync_copy(z80_h, degi_v)
    zs = pl.ds(s * (NR // NS), NR // NS)
    pltpu.sync_copy(z80_h.at[pl.ds(0, NR // NS)], dego_sh.at[zs])
    pltpu.sync_copy(z80_h.at[pl.ds(0, NR // NS)], degi_sh.at[zs])
    plsc.subcore_barrier()
    ones16 = jnp.full((L,), 1.0, jnp.float32)
    c127 = jnp.full((L,), 127, jnp.int32)

    def chunk(j, carry):
        for k in range(CH // L):
            sl = pl.ds(j * CH + k * L, L)
            sv = srcv[sl]
            dv = dstv[sl]
            plsc.addupdate_scatter(
                dego_v, [lax.shift_right_logical(sv, 7), sv & c127], ones16)
            plsc.addupdate_scatter(
                degi_v, [lax.shift_right_logical(dv, 7), dv & c127], ones16)
        return carry

    lax.fori_loop(0, IT, chunk, 0)
    pltpu.sync_copy(dego_v, dego_sh.at[idn_v], add=True)
    pltpu.sync_copy(degi_v, degi_sh.at[idn_v], add=True)
    plsc.subcore_barrier()
    pltpu.sync_copy(dego_sh.at[zs], out_h.at[c, 0, zs])
    pltpu.sync_copy(degi_sh.at[zs], out_h.at[c, 1, zs])


@functools.partial(
    pl.kernel,
    out_type=jax.ShapeDtypeStruct((NC, 2, NR, 128), jnp.float32),
    mesh=_mesh,
    compiler_params=pltpu.CompilerParams(
        needs_layout_passes=False, use_tc_tiling_on_sc=False),
    scratch_types=[
        pltpu.VMEM((EPT,), jnp.int32),
        pltpu.VMEM((EPT,), jnp.int32),
        pltpu.VMEM((NR, 128), jnp.float32),
        pltpu.VMEM((NR, 128), jnp.float32),
        pltpu.VMEM((NR,), jnp.int32),
        pltpu.VMEM_SHARED((NR, 128), jnp.float32),
        pltpu.VMEM_SHARED((NR, 128), jnp.float32),
    ],
)
def _deg_kernel(ei_h, z80_h, idn_h, out_h, *scratch):
    _deg_body(ei_h, z80_h, idn_h, out_h, *scratch)


G = 2
NG = IT2 // G


def _agg_body(m_h, nd_h, s2a_h, ei_h, zrow_h, znp_h, agg_out, c_out,
              idxv, rbuf2, tbl, agg_sh, sg0, sg1, ss0, ss1):
    c = lax.axis_index("c")
    s = lax.axis_index("s")
    wid = c * NS + s
    es = pl.ds(s * EPS, EPS)
    pltpu.sync_copy(s2a_h.at[es], idxv.at[pl.ds(0, EPS)])
    pltpu.sync_copy(ei_h.at[1, es], idxv.at[pl.ds(EPS, EPS)])
    pltpu.sync_copy(nd_h, tbl.at[pl.ds(0, NP)])
    pltpu.sync_copy(znp_h, tbl.at[pl.ds(NP, NP)])
    pltpu.sync_copy(zrow_h, agg_sh.at[pl.ds(s * STRIPE, STRIPE)])
    plsc.subcore_barrier()

    mc = m_h.at[pl.ds(c, 2 * N - 1)]
    npv = jnp.full((L,), NP, jnp.int32)
    GB = G * CH

    def rb(p, k):
        return rbuf2.at[pl.ds((p * G + k) * CH, CH)]

    def gidx(j):
        return idxv.at[pl.ds(j * CH, CH)]

    def didx(j):
        return idxv.at[pl.ds(EPS + j * CH, CH)]

    def fire_gathers(g, p, sem):
        for k in range(G):
            pltpu.async_copy(mc.at[gidx(g * G + k)], rb(p, k), sem)

    def drain_group(p, sem):
        pltpu.make_async_copy(
            mc.at[pl.ds(0, GB)], rbuf2.at[pl.ds(p * GB, GB)], sem).wait()

    def fire_scatters(g, p, sem):
        for k in range(G):
            pltpu.async_copy(
                rb(p, k), agg_sh.at[didx(g * G + k)], sem, add=True)

    def vpu_c(g):
        for k in range(G):
            j = g * G + k
            for kk in range(CH // L):
                vals = plsc.load_gather(
                    tbl, [idxv[pl.ds(EPS + j * CH + kk * L, L)]])
                plsc.addupdate_scatter(
                    tbl,
                    [lax.shift_right_logical(
                        idxv[pl.ds(j * CH + kk * L, L)], 1) + npv], vals)

    fire_gathers(0, 0, sg0)

    def outer(i, carry):
        g0 = 2 * i
        fire_gathers(g0 + 1, 1, sg1)
        drain_group(0, sg0)
        fire_scatters(g0, 0, ss0)
        vpu_c(g0)
        drain_group(0, ss0)
        fire_gathers(g0 + 2, 0, sg0)
        drain_group(1, sg1)
        fire_scatters(g0 + 1, 1, ss1)
        vpu_c(g0 + 1)
        drain_group(1, ss1)
        return carry

    lax.fori_loop(0, NG // 2, outer, 0)
    drain_group(0, sg0)
    fire_scatters(NG - 1, 0, ss0)
    vpu_c(NG - 1)
    drain_group(0, ss0)

    pltpu.sync_copy(tbl.at[pl.ds(NP, NP)], c_out.at[wid])
    plsc.subcore_barrier()
    sl = pl.ds(s * STRIPE, STRIPE)
    pltpu.sync_copy(agg_sh.at[sl], agg_out.at[c, sl])


@functools.partial(
    pl.kernel,
    out_type=(
        jax.ShapeDtypeStruct((NC, NP, DH), jnp.float32),
        jax.ShapeDtypeStruct((NW, NP), jnp.float32),
    ),
    mesh=_mesh,
    compiler_params=pltpu.CompilerParams(
        needs_layout_passes=False, use_tc_tiling_on_sc=False),
    scratch_types=[
        pltpu.VMEM((2 * EPS,), jnp.int32),
        pltpu.VMEM((2 * G * CH, DH), jnp.float32),
        pltpu.VMEM((2 * NP,), jnp.float32),
        pltpu.VMEM_SHARED((NP, DH), jnp.float32),
        pltpu.SemaphoreType.DMA,
        pltpu.SemaphoreType.DMA,
        pltpu.SemaphoreType.DMA,
        pltpu.SemaphoreType.DMA,
    ],
)
def _agg_kernel(m_h, nd_h, s2a_h, ei_h, zrow_h, znp_h,
                agg_out, c_out, *scratch):
    _agg_body(m_h, nd_h, s2a_h, ei_h, zrow_h, znp_h,
              agg_out, c_out, *scratch)


def _mm_body(h_ref, w_ref, o_ref, sh_ref):
    o_ref[...] = jnp.dot(
        h_ref[...], w_ref[...], preferred_element_type=jnp.float32)
    sh_ref[...] = jnp.sum(h_ref[...], axis=0, keepdims=True)


def _scale_body(xw_ref, ns_ref, o_ref):
    o_ref[...] = xw_ref[...] * ns_ref[...]


def _fin_body(agg_ref, c_ref, nd_ref, ns_ref, sh_ref, b1_ref,
              w2_ref, b2_ref, wl_ref, bl_ref, o_ref):
    agg = jnp.concatenate(
        [agg_ref[0, :N, :], agg_ref[1, :N, :]], axis=1)
    x = jnp.maximum(agg * nd_ref[...] + b1_ref[...], 0.0)
    c_row = 0.5 * jnp.sum(c_ref[...], axis=0, keepdims=True)[:, :N]
    w_row = ns_ref[...] * c_row
    s = jnp.dot(w_row, x, preferred_element_type=jnp.float32)
    sh = sh_ref[...]
    o_ref[...] = (
        jnp.dot(s, w2_ref[...], preferred_element_type=jnp.float32)
        + jnp.dot(sh, wl_ref[...], preferred_element_type=jnp.float32)
    ) * (1.0 / N) + b2_ref[...] + bl_ref[...]


def kernel(h, edge_index, W1, b1, W2, b2, Wl, bl):
    ei = edge_index.astype(jnp.int32)
    s2a = ei[0] + ei[0]
    zrow = jnp.zeros((STRIPE, DH), jnp.float32)
    znp = jnp.zeros((NP,), jnp.float32)
    z80 = jnp.zeros((NR, 128), jnp.float32)
    idn = jnp.arange(NR, dtype=jnp.int32)

    xw, sh = pl.pallas_call(
        _mm_body,
        out_shape=(
            jax.ShapeDtypeStruct((N, D), jnp.float32),
            jax.ShapeDtypeStruct((1, D), jnp.float32),
        ),
    )(h, W1)

    deg_parts = _deg_kernel(ei, z80, idn)
    deg = deg_parts.sum(axis=0).reshape(2, NP)[:, :N]
    norm = jnp.where(deg > 0, lax.rsqrt(jnp.maximum(deg, 1.0)), 0.0)
    ns_col = norm[0].reshape(N, 1)
    nd_col = norm[1].reshape(N, 1)
    nd_flat = jnp.pad(norm[1], (0, NP - N))

    m = pl.pallas_call(
        _scale_body,
        out_shape=jax.ShapeDtypeStruct((N, D), jnp.float32),
    )(xw, ns_col)
    m2 = m.reshape(2 * N, DH)

    agg_pp, c_pp = _agg_kernel(m2, nd_flat, s2a, ei, zrow, znp)
    ns_row = norm[0].reshape(1, N)

    out = pl.pallas_call(
        _fin_body,
        out_shape=jax.ShapeDtypeStruct((1, D), jnp.float32),
    )(agg_pp, c_pp, nd_col, ns_row, sh,
      b1.reshape(1, D), W2, b2.reshape(1, D), Wl, bl.reshape(1, D))
    return out


# speedup vs baseline: 1.1946x; 1.0534x over previous
import functools

import jax
import jax.numpy as jnp
from jax import lax
from jax.experimental import pallas as pl
from jax.experimental.pallas import tpu as pltpu
from jax.experimental.pallas import tpu_sc as plsc

N = 10000
NP = 10240
E = 320000
D = 128

NC = 2
NS = 16
NW = NC * NS
L = 16
EPT = E // NW
CH = 80
IT = EPT // CH
DH = D // NC
EPS = E // NS
IT2 = EPS // CH
STRIPE = NP // NS
NR = NP // 128

_mesh = plsc.VectorSubcoreMesh(core_axis_name="c", subcore_axis_name="s")


def _deg_body(ei_h, z80_h, idn_h, out_h,
              srcv, dstv, dego_v, degi_v, idn_v, dego_sh, degi_sh):
    c = lax.axis_index("c")
    s = lax.axis_index("s")
    wid = c * NS + s
    pltpu.sync_copy(ei_h.at[0, pl.ds(wid * EPT, EPT)], srcv)
    pltpu.sync_copy(ei_h.at[1, pl.ds(wid * EPT, EPT)], dstv)
    pltpu.sync_copy(idn_h, idn_v)
    pltpu.sync_copy(z80_h, dego_v)
    pltpu.sync_copy(z80_h, degi_v)
    zs = pl.ds(s * (NR // NS), NR // NS)
    pltpu.sync_copy(z80_h.at[pl.ds(0, NR // NS)], dego_sh.at[zs])
    pltpu.sync_copy(z80_h.at[pl.ds(0, NR // NS)], degi_sh.at[zs])
    plsc.subcore_barrier()
    ones16 = jnp.full((L,), 1.0, jnp.float32)
    c127 = jnp.full((L,), 127, jnp.int32)

    def chunk(j, carry):
        for k in range(CH // L):
            sl = pl.ds(j * CH + k * L, L)
            sv = srcv[sl]
            dv = dstv[sl]
            plsc.addupdate_scatter(
                dego_v, [lax.shift_right_logical(sv, 7), sv & c127], ones16)
            plsc.addupdate_scatter(
                degi_v, [lax.shift_right_logical(dv, 7), dv & c127], ones16)
        return carry

    lax.fori_loop(0, IT, chunk, 0)
    pltpu.sync_copy(dego_v, dego_sh.at[idn_v], add=True)
    pltpu.sync_copy(degi_v, degi_sh.at[idn_v], add=True)
    plsc.subcore_barrier()
    pltpu.sync_copy(dego_sh.at[zs], out_h.at[c, 0, zs])
    pltpu.sync_copy(degi_sh.at[zs], out_h.at[c, 1, zs])


@functools.partial(
    pl.kernel,
    out_type=jax.ShapeDtypeStruct((NC, 2, NR, 128), jnp.float32),
    mesh=_mesh,
    compiler_params=pltpu.CompilerParams(
        needs_layout_passes=False, use_tc_tiling_on_sc=False),
    scratch_types=[
        pltpu.VMEM((EPT,), jnp.int32),
        pltpu.VMEM((EPT,), jnp.int32),
        pltpu.VMEM((NR, 128), jnp.float32),
        pltpu.VMEM((NR, 128), jnp.float32),
        pltpu.VMEM((NR,), jnp.int32),
        pltpu.VMEM_SHARED((NR, 128), jnp.float32),
        pltpu.VMEM_SHARED((NR, 128), jnp.float32),
    ],
)
def _deg_kernel(ei_h, z80_h, idn_h, out_h, *scratch):
    _deg_body(ei_h, z80_h, idn_h, out_h, *scratch)


G = 2
NG = IT2 // G


def _agg_body(m_h, nd_h, s2a_h, ei_h, zrow_h, znp_h, idn2_h, agg_out, c_out,
              idxv, rbuf2, tbl, idn2_v, agg_sh, sg0, sg1, ss0, ss1):
    c = lax.axis_index("c")
    s = lax.axis_index("s")
    wid = c * NS + s
    es = pl.ds(s * EPS, EPS)
    pltpu.sync_copy(s2a_h.at[es], idxv.at[pl.ds(0, EPS)])
    pltpu.sync_copy(ei_h.at[1, es], idxv.at[pl.ds(EPS, EPS)])
    pltpu.sync_copy(idn2_h.at[c, pl.ds(s * STRIPE, STRIPE)], idn2_v)
    pltpu.sync_copy(nd_h, tbl.at[pl.ds(0, NP)])
    pltpu.sync_copy(znp_h, tbl.at[pl.ds(NP, NP)])
    pltpu.sync_copy(zrow_h, agg_sh.at[pl.ds(s * STRIPE, STRIPE)])
    plsc.subcore_barrier()

    mc = m_h.at[pl.ds(c, 2 * N - 1)]
    npv = jnp.full((L,), NP, jnp.int32)
    GB = G * CH

    def rb(p, k):
        return rbuf2.at[pl.ds((p * G + k) * CH, CH)]

    def gidx(j):
        return idxv.at[pl.ds(j * CH, CH)]

    def didx(j):
        return idxv.at[pl.ds(EPS + j * CH, CH)]

    def fire_gathers(g, p, sem):
        for k in range(G):
            pltpu.async_copy(mc.at[gidx(g * G + k)], rb(p, k), sem)

    def drain_group(p, sem):
        pltpu.make_async_copy(
            mc.at[pl.ds(0, GB)], rbuf2.at[pl.ds(p * GB, GB)], sem).wait()

    def fire_scatters(g, p, sem):
        for k in range(G):
            pltpu.async_copy(
                rb(p, k), agg_sh.at[didx(g * G + k)], sem, add=True)

    def vpu_c(g):
        for k in range(G):
            j = g * G + k
            for kk in range(CH // L):
                vals = plsc.load_gather(
                    tbl, [idxv[pl.ds(EPS + j * CH + kk * L, L)]])
                plsc.addupdate_scatter(
                    tbl,
                    [lax.shift_right_logical(
                        idxv[pl.ds(j * CH + kk * L, L)], 1) + npv], vals)

    fire_gathers(0, 0, sg0)

    def outer(i, carry):
        g0 = 2 * i
        fire_gathers(g0 + 1, 1, sg1)
        drain_group(0, sg0)
        fire_scatters(g0, 0, ss0)
        vpu_c(g0)
        drain_group(0, ss0)
        fire_gathers(g0 + 2, 0, sg0)
        drain_group(1, sg1)
        fire_scatters(g0 + 1, 1, ss1)
        vpu_c(g0 + 1)
        drain_group(1, ss1)
        return carry

    lax.fori_loop(0, NG // 2, outer, 0)
    drain_group(0, sg0)
    fire_scatters(NG - 1, 0, ss0)
    vpu_c(NG - 1)
    drain_group(0, ss0)

    pltpu.sync_copy(tbl.at[pl.ds(NP, NP)], c_out.at[wid])
    plsc.subcore_barrier()
    for k in range(STRIPE // 128):
        pltpu.sync_copy(
            agg_sh.at[pl.ds(s * STRIPE + k * 128, 128)],
            rbuf2.at[pl.ds(0, 128)])
        pltpu.sync_copy(
            rbuf2.at[pl.ds(0, 128)],
            agg_out.at[idn2_v.at[pl.ds(k * 128, 128)]])


@functools.partial(
    pl.kernel,
    out_type=(
        jax.ShapeDtypeStruct((2 * NP, DH), jnp.float32),
        jax.ShapeDtypeStruct((NW, NP), jnp.float32),
    ),
    mesh=_mesh,
    compiler_params=pltpu.CompilerParams(
        needs_layout_passes=False, use_tc_tiling_on_sc=False),
    scratch_types=[
        pltpu.VMEM((2 * EPS,), jnp.int32),
        pltpu.VMEM((2 * G * CH, DH), jnp.float32),
        pltpu.VMEM((2 * NP,), jnp.float32),
        pltpu.VMEM((STRIPE,), jnp.int32),
        pltpu.VMEM_SHARED((NP, DH), jnp.float32),
        pltpu.SemaphoreType.DMA,
        pltpu.SemaphoreType.DMA,
        pltpu.SemaphoreType.DMA,
        pltpu.SemaphoreType.DMA,
    ],
)
def _agg_kernel(m_h, nd_h, s2a_h, ei_h, zrow_h, znp_h, idn2_h,
                agg_out, c_out, *scratch):
    _agg_body(m_h, nd_h, s2a_h, ei_h, zrow_h, znp_h, idn2_h,
              agg_out, c_out, *scratch)


def _mm_body(h_ref, w_ref, o_ref, sh_ref):
    o_ref[...] = jnp.dot(
        h_ref[...], w_ref[...], preferred_element_type=jnp.float32)
    sh_ref[...] = jnp.sum(h_ref[...], axis=0, keepdims=True)


def _scale_body(xw_ref, ns_ref, o_ref):
    o_ref[...] = xw_ref[...] * ns_ref[...]


def _fin_body(agg_ref, c_ref, nd_ref, ns_ref, sh_ref, b1_ref,
              w2_ref, b2_ref, wl_ref, bl_ref, o_ref):
    x = jnp.maximum(agg_ref[:N, :] * nd_ref[...] + b1_ref[...], 0.0)
    c_row = 0.5 * jnp.sum(c_ref[...], axis=0, keepdims=True)[:, :N]
    w_row = ns_ref[...] * c_row
    s = jnp.dot(w_row, x, preferred_element_type=jnp.float32)
    sh = sh_ref[...]
    o_ref[...] = (
        jnp.dot(s, w2_ref[...], preferred_element_type=jnp.float32)
        + jnp.dot(sh, wl_ref[...], preferred_element_type=jnp.float32)
    ) * (1.0 / N) + b2_ref[...] + bl_ref[...]


def kernel(h, edge_index, W1, b1, W2, b2, Wl, bl):
    ei = edge_index.astype(jnp.int32)
    s2a = ei[0] + ei[0]
    zrow = jnp.zeros((STRIPE, DH), jnp.float32)
    znp = jnp.zeros((NP,), jnp.float32)
    z80 = jnp.zeros((NR, 128), jnp.float32)
    idn = jnp.arange(NR, dtype=jnp.int32)
    iv = jnp.arange(NP, dtype=jnp.int32)
    idn2 = jnp.stack([iv + iv, iv + iv + 1])

    xw, sh = pl.pallas_call(
        _mm_body,
        out_shape=(
            jax.ShapeDtypeStruct((N, D), jnp.float32),
            jax.ShapeDtypeStruct((1, D), jnp.float32),
        ),
    )(h, W1)

    deg_parts = _deg_kernel(ei, z80, idn)
    deg = deg_parts.sum(axis=0).reshape(2, NP)[:, :N]
    norm = jnp.where(deg > 0, lax.rsqrt(jnp.maximum(deg, 1.0)), 0.0)
    ns_col = norm[0].reshape(N, 1)
    nd_col = norm[1].reshape(N, 1)
    nd_flat = jnp.pad(norm[1], (0, NP - N))

    m = pl.pallas_call(
        _scale_body,
        out_shape=jax.ShapeDtypeStruct((N, D), jnp.float32),
    )(xw, ns_col)
    m2 = m.reshape(2 * N, DH)

    agg_pp, c_pp = _agg_kernel(m2, nd_flat, s2a, ei, zrow, znp, idn2)
    agg128 = agg_pp.reshape(NP, D)
    ns_row = norm[0].reshape(1, N)

    out = pl.pallas_call(
        _fin_body,
        out_shape=jax.ShapeDtypeStruct((1, D), jnp.float32),
    )(agg128, c_pp, nd_col, ns_row, sh,
      b1.reshape(1, D), W2, b2.reshape(1, D), Wl, bl.reshape(1, D))
    return out
